# Initial kernel scaffold; baseline (speedup 1.0000x reference)
import jax
import jax.numpy as jnp
from jax.experimental import pallas as pl


def kernel(x_nchw, layer1_w, layer1_scale, layer1_shift, layer2_w, layer2_scale, layer2_shift, layer3_w, layer3_scale, layer3_shift, layer4_w, layer4_scale, layer4_shift, layer5_w, layer5_scale, layer5_shift, fc1_w, fc1_scale, fc1_shift):
    raise NotImplementedError("write your pallas kernel here")


# speedup vs baseline: 2.9677x; 2.9677x over previous
import functools

import jax
import jax.numpy as jnp
from jax.experimental import pallas as pl
from jax.experimental.pallas import tpu as pltpu

VMEM_LIMIT = 64 * 1024 * 1024


def _pick_nb(B, nb):
    nb = min(nb, B)
    while B % nb:
        nb //= 2
    return max(nb, 1)


def _l1_kernel(xp_ref, wb_ref, scale_ref, shift_ref, out_ref):
    NB = xp_ref.shape[0]
    acc = jnp.zeros((NB * 32, 2048), jnp.float32)
    for p in range(6):
        lhs = xp_ref[:, 2 * p:2 * p + 32, :].reshape(NB * 32, 256)
        acc = acc + jnp.dot(lhs, wb_ref[p], preferred_element_type=jnp.float32)
    z = jnp.maximum(acc * scale_ref[...] + shift_ref[...], 0.0)
    z = z.reshape(NB, 32, 32, 64)
    z = jnp.max(z.reshape(NB, 16, 2, 32, 64), axis=2)
    z = jnp.max(z.reshape(NB, 16, 16, 2, 64), axis=3)
    out_ref[...] = z.reshape(NB, 256, 64).astype(out_ref.dtype)


def _build_l1_band(layer1_w):
    w4 = layer1_w[:363].astype(jnp.float32).reshape(3, 11, 11, 64)
    wi = jnp.arange(42)
    wo = jnp.arange(32)
    kw = jnp.arange(11)
    band = (wi[None, :, None] - wo[None, None, :] == kw[:, None, None])
    band = band.astype(jnp.float32)
    wb = jnp.einsum("kiw,cdkn->dicwn", band, w4)
    wb = wb.reshape(11, 126, 2048)
    wb = jnp.pad(wb, ((0, 1), (0, 2), (0, 0)))
    return wb.reshape(6, 256, 2048).astype(jnp.bfloat16)


def _layer1(x_nchw, layer1_w, scale, shift):
    B = x_nchw.shape[0]
    x = jnp.transpose(x_nchw, (0, 2, 3, 1)).astype(jnp.bfloat16)
    x = jnp.pad(x, ((0, 0), (1, 1), (1, 1), (0, 0)))
    x2 = jnp.pad(x.reshape(B, 42, 126), ((0, 0), (0, 0), (0, 2)))
    xshift = jnp.pad(x2[:, 1:, :], ((0, 0), (0, 1), (0, 0)))
    xpair = jnp.concatenate([x2, xshift], axis=2)
    wb = _build_l1_band(layer1_w)
    s = jnp.tile(scale, (1, 32))
    t = jnp.tile(shift, (1, 32))
    NB = _pick_nb(B, 64)
    return pl.pallas_call(
        _l1_kernel,
        out_shape=jax.ShapeDtypeStruct((B, 256, 64), jnp.bfloat16),
        grid=(B // NB,),
        in_specs=[
            pl.BlockSpec((NB, 42, 256), lambda i: (i, 0, 0)),
            pl.BlockSpec((6, 256, 2048), lambda i: (0, 0, 0)),
            pl.BlockSpec((1, 2048), lambda i: (0, 0)),
            pl.BlockSpec((1, 2048), lambda i: (0, 0)),
        ],
        out_specs=pl.BlockSpec((NB, 256, 64), lambda i: (i, 0, 0)),
        compiler_params=pltpu.CompilerParams(
            dimension_semantics=("parallel",),
            vmem_limit_bytes=VMEM_LIMIT,
        ),
    )(xpair, wb, s, t)


def _conv_kernel(x_ref, w_ref, scale_ref, shift_ref, out_ref, *, Wp2, Ho, Wo):
    NB, _, Cin = x_ref.shape
    Cout = w_ref.shape[2]
    M2 = Ho * Wp2
    acc = jnp.zeros((NB * M2, Cout), jnp.float32)
    t = 0
    for dh in range(3):
        for dw in range(3):
            off = dh * Wp2 + dw
            lhs = x_ref[:, off:off + M2, :].reshape(NB * M2, Cin)
            acc = acc + jnp.dot(lhs, w_ref[t], preferred_element_type=jnp.float32)
            t += 1
    z = jnp.maximum(acc * scale_ref[...] + shift_ref[...], 0.0)
    z = z.reshape(NB, Ho, Wp2, Cout)[:, :, :Wo, :]
    Hp, Wp = Ho // 2, Wo // 2
    z = jnp.max(z.reshape(NB, Hp, 2, Wo, Cout), axis=2)
    z = jnp.max(z.reshape(NB, Hp, Wp, 2, Cout), axis=3)
    out_ref[...] = z.reshape(NB, Hp * Wp, Cout).astype(out_ref.dtype)


def _taps(w_flat, cin, cout):
    return w_flat[:cin * 9].reshape(cin, 9, cout).transpose(1, 0, 2)


def _pad_flat(y, Hp, Wp, C, R):
    B = y.shape[0]
    y = y.reshape(B, Hp, Wp, C)
    y = jnp.pad(y, ((0, 0), (1, 1), (1, 1), (0, 0)))
    y = y.reshape(B, (Hp + 2) * (Wp + 2), C)
    extra = R - (Hp + 2) * (Wp + 2)
    if extra:
        y = jnp.pad(y, ((0, 0), (0, extra), (0, 0)))
    return y


def _conv_layer(x_flat, w_taps, scale, shift, *, nb, Wp2, Ho, Wo, R):
    B, _, Cin = x_flat.shape
    Cout = w_taps.shape[2]
    Hp, Wp = Ho // 2, Wo // 2
    NB = _pick_nb(B, nb)
    return pl.pallas_call(
        functools.partial(_conv_kernel, Wp2=Wp2, Ho=Ho, Wo=Wo),
        out_shape=jax.ShapeDtypeStruct((B, Hp * Wp, Cout), jnp.bfloat16),
        grid=(B // NB,),
        in_specs=[
            pl.BlockSpec((NB, R, Cin), lambda i: (i, 0, 0)),
            pl.BlockSpec((9, Cin, Cout), lambda i: (0, 0, 0)),
            pl.BlockSpec((1, Cout), lambda i: (0, 0)),
            pl.BlockSpec((1, Cout), lambda i: (0, 0)),
        ],
        out_specs=pl.BlockSpec((NB, Hp * Wp, Cout), lambda i: (i, 0, 0)),
        compiler_params=pltpu.CompilerParams(
            dimension_semantics=("parallel",),
            vmem_limit_bytes=VMEM_LIMIT,
        ),
    )(x_flat, w_taps, scale, shift)


def _l5_fc_kernel(x_ref, w_ref, s5_ref, b5_ref, wfc_ref, sfc_ref, bfc_ref,
                  out_ref):
    NB = x_ref.shape[0]
    acc = jnp.zeros((NB * 8, 1024), jnp.float32)
    t = 0
    for dh in range(3):
        for dw in range(3):
            off = dh * 4 + dw
            lhs = x_ref[:, off:off + 8, :].reshape(NB * 8, 256)
            acc = acc + jnp.dot(lhs, w_ref[t], preferred_element_type=jnp.float32)
            t += 1
    z = jnp.maximum(acc * s5_ref[...] + b5_ref[...], 0.0)
    z = jnp.max(z.reshape(NB, 2, 4, 1024), axis=1)
    z = jnp.max(z[:, :2, :], axis=1)
    f = jnp.dot(z.astype(jnp.bfloat16), wfc_ref[...],
                preferred_element_type=jnp.float32)
    f = jnp.maximum(f * sfc_ref[...] + bfc_ref[...], 0.0)
    out_ref[...] = f[:, :3600]


def _layer5_fc(x_flat, w_taps, s5, b5, wfc, sfc, bfc):
    B = x_flat.shape[0]
    NB = _pick_nb(B, 128)
    return pl.pallas_call(
        _l5_fc_kernel,
        out_shape=jax.ShapeDtypeStruct((B, 3600), jnp.float32),
        grid=(B // NB,),
        in_specs=[
            pl.BlockSpec((NB, 24, 256), lambda i: (i, 0, 0)),
            pl.BlockSpec((9, 256, 1024), lambda i: (0, 0, 0)),
            pl.BlockSpec((1, 1024), lambda i: (0, 0)),
            pl.BlockSpec((1, 1024), lambda i: (0, 0)),
            pl.BlockSpec((1024, 3840), lambda i: (0, 0)),
            pl.BlockSpec((1, 3840), lambda i: (0, 0)),
            pl.BlockSpec((1, 3840), lambda i: (0, 0)),
        ],
        out_specs=pl.BlockSpec((NB, 3600), lambda i: (i, 0)),
        compiler_params=pltpu.CompilerParams(
            dimension_semantics=("parallel",),
            vmem_limit_bytes=VMEM_LIMIT,
        ),
    )(x_flat, w_taps, s5, b5, wfc, sfc, bfc)


def kernel(x_nchw,
           layer1_w, layer1_scale, layer1_shift,
           layer2_w, layer2_scale, layer2_shift,
           layer3_w, layer3_scale, layer3_shift,
           layer4_w, layer4_scale, layer4_shift,
           layer5_w, layer5_scale, layer5_shift,
           fc1_w, fc1_scale, fc1_shift):
    y = _layer1(x_nchw, layer1_w, layer1_scale, layer1_shift)
    y = _conv_layer(_pad_flat(y, 16, 16, 64, 328),
                    _taps(layer2_w, 64, 128), layer2_scale, layer2_shift,
                    nb=64, Wp2=18, Ho=16, Wo=16, R=328)
    y = _conv_layer(_pad_flat(y, 8, 8, 128, 104),
                    _taps(layer3_w, 128, 256), layer3_scale, layer3_shift,
                    nb=128, Wp2=10, Ho=8, Wo=8, R=104)
    y = _conv_layer(_pad_flat(y, 4, 4, 256, 40),
                    _taps(layer4_w, 256, 256), layer4_scale, layer4_shift,
                    nb=256, Wp2=6, Ho=4, Wo=4, R=40)
    return _layer5_fc(_pad_flat(y, 2, 2, 256, 24),
                      _taps(layer5_w, 256, 1024), layer5_scale, layer5_shift,
                      fc1_w, fc1_scale, fc1_shift)


# speedup vs baseline: 3.2895x; 1.1084x over previous
import functools

import jax
import jax.numpy as jnp
from jax.experimental import pallas as pl
from jax.experimental.pallas import tpu as pltpu

VMEM_LIMIT = 64 * 1024 * 1024


def _pick_nb(B, nb):
    nb = min(nb, B)
    while B % nb:
        nb //= 2
    return max(nb, 1)


def _l1_kernel(xp_ref, wb_ref, scale_ref, shift_ref, out_ref):
    NB = xp_ref.shape[0]
    acc = jnp.zeros((NB * 32, 2048), jnp.float32)
    for p in range(6):
        lhs = xp_ref[:, 2 * p:2 * p + 32, :].reshape(NB * 32, 256)
        acc = acc + jnp.dot(lhs, wb_ref[p], preferred_element_type=jnp.float32)
    z = jnp.maximum(acc * scale_ref[...] + shift_ref[...], 0.0)
    z = z.reshape(NB, 32, 32, 64)
    z = jnp.max(z.reshape(NB, 16, 2, 32, 64), axis=2)
    z = jnp.max(z.reshape(NB, 16, 16, 2, 64), axis=3)
    out_ref[...] = z.reshape(NB, 256, 64).astype(out_ref.dtype)


def _build_l1_band(layer1_w):
    w4 = layer1_w[:363].astype(jnp.float32).reshape(3, 11, 11, 64)
    wi = jnp.arange(42)
    wo = jnp.arange(32)
    kw = jnp.arange(11)
    band = (wi[None, :, None] - wo[None, None, :] == kw[:, None, None])
    band = band.astype(jnp.float32)
    wb = jnp.einsum("kiw,cdkn->dicwn", band, w4)
    wb = wb.reshape(11, 126, 2048)
    wb = jnp.pad(wb, ((0, 1), (0, 2), (0, 0)))
    return wb.reshape(6, 256, 2048).astype(jnp.bfloat16)


def _layer1(x_nchw, layer1_w, scale, shift):
    B = x_nchw.shape[0]
    x = jnp.transpose(x_nchw, (0, 2, 3, 1)).astype(jnp.bfloat16)
    x = jnp.pad(x, ((0, 0), (1, 1), (1, 1), (0, 0)))
    x2 = jnp.pad(x.reshape(B, 42, 126), ((0, 0), (0, 0), (0, 2)))
    xshift = jnp.pad(x2[:, 1:, :], ((0, 0), (0, 1), (0, 0)))
    xpair = jnp.concatenate([x2, xshift], axis=2)
    wb = _build_l1_band(layer1_w)
    s = jnp.tile(scale, (1, 32))
    t = jnp.tile(shift, (1, 32))
    NB = _pick_nb(B, 64)
    return pl.pallas_call(
        _l1_kernel,
        out_shape=jax.ShapeDtypeStruct((B, 256, 64), jnp.bfloat16),
        grid=(B // NB,),
        in_specs=[
            pl.BlockSpec((NB, 42, 256), lambda i: (i, 0, 0)),
            pl.BlockSpec((6, 256, 2048), lambda i: (0, 0, 0)),
            pl.BlockSpec((1, 2048), lambda i: (0, 0)),
            pl.BlockSpec((1, 2048), lambda i: (0, 0)),
        ],
        out_specs=pl.BlockSpec((NB, 256, 64), lambda i: (i, 0, 0)),
        compiler_params=pltpu.CompilerParams(
            dimension_semantics=("parallel",),
            vmem_limit_bytes=VMEM_LIMIT,
        ),
    )(xpair, wb, s, t)


def _conv3_kernel(x_ref, w_ref, scale_ref, shift_ref, out_ref, *, Wo, Ho):
    NB, _, K3 = x_ref.shape
    Cout = w_ref.shape[2]
    M2 = Ho * Wo
    acc = jnp.zeros((NB * M2, Cout), jnp.float32)
    for dh in range(3):
        lhs = x_ref[:, dh * Wo:dh * Wo + M2, :].reshape(NB * M2, K3)
        acc = acc + jnp.dot(lhs, w_ref[dh], preferred_element_type=jnp.float32)
    z = jnp.maximum(acc * scale_ref[...] + shift_ref[...], 0.0)
    Hp, Wp = Ho // 2, Wo // 2
    z = jnp.max(z.reshape(NB, Hp, 2, Wo, Cout), axis=2)
    z = jnp.max(z.reshape(NB, Hp, Wp, 2, Cout), axis=3)
    out_ref[...] = z.reshape(NB, Hp * Wp, Cout).astype(out_ref.dtype)


def _kw3_input(y, Hin, Win, C):
    B = y.shape[0]
    y = y.reshape(B, Hin, Win, C)
    y = jnp.pad(y, ((0, 0), (1, 1), (1, 1), (0, 0)))
    xk = jnp.concatenate([y[:, :, dw:dw + Win, :] for dw in range(3)], axis=3)
    return xk.reshape(B, (Hin + 2) * Win, 3 * C)


def _w_kw3(w_flat, cin, cout):
    w = w_flat[:cin * 9].reshape(cin, 3, 3, cout)
    return w.transpose(1, 2, 0, 3).reshape(3, 3 * cin, cout)


def _conv3_layer(xk, w3, scale, shift, *, nb, Ho, Wo):
    B, R, K3 = xk.shape
    Cout = w3.shape[2]
    Hp, Wp = Ho // 2, Wo // 2
    NB = _pick_nb(B, nb)
    return pl.pallas_call(
        functools.partial(_conv3_kernel, Wo=Wo, Ho=Ho),
        out_shape=jax.ShapeDtypeStruct((B, Hp * Wp, Cout), jnp.bfloat16),
        grid=(B // NB,),
        in_specs=[
            pl.BlockSpec((NB, R, K3), lambda i: (i, 0, 0)),
            pl.BlockSpec((3, K3, Cout), lambda i: (0, 0, 0)),
            pl.BlockSpec((1, Cout), lambda i: (0, 0)),
            pl.BlockSpec((1, Cout), lambda i: (0, 0)),
        ],
        out_specs=pl.BlockSpec((NB, Hp * Wp, Cout), lambda i: (i, 0, 0)),
        compiler_params=pltpu.CompilerParams(
            dimension_semantics=("parallel",),
            vmem_limit_bytes=VMEM_LIMIT,
        ),
    )(xk, w3, scale, shift)


def _conv_kernel(x_ref, w_ref, scale_ref, shift_ref, out_ref, *, Wp2, Ho, Wo):
    NB, _, Cin = x_ref.shape
    Cout = w_ref.shape[2]
    M2 = Ho * Wp2
    acc = jnp.zeros((NB * M2, Cout), jnp.float32)
    t = 0
    for dh in range(3):
        for dw in range(3):
            off = dh * Wp2 + dw
            lhs = x_ref[:, off:off + M2, :].reshape(NB * M2, Cin)
            acc = acc + jnp.dot(lhs, w_ref[t], preferred_element_type=jnp.float32)
            t += 1
    z = jnp.maximum(acc * scale_ref[...] + shift_ref[...], 0.0)
    z = z.reshape(NB, Ho, Wp2, Cout)[:, :, :Wo, :]
    Hp, Wp = Ho // 2, Wo // 2
    z = jnp.max(z.reshape(NB, Hp, 2, Wo, Cout), axis=2)
    z = jnp.max(z.reshape(NB, Hp, Wp, 2, Cout), axis=3)
    out_ref[...] = z.reshape(NB, Hp * Wp, Cout).astype(out_ref.dtype)


def _taps(w_flat, cin, cout):
    return w_flat[:cin * 9].reshape(cin, 9, cout).transpose(1, 0, 2)


def _pad_flat(y, Hp, Wp, C, R):
    B = y.shape[0]
    y = y.reshape(B, Hp, Wp, C)
    y = jnp.pad(y, ((0, 0), (1, 1), (1, 1), (0, 0)))
    y = y.reshape(B, (Hp + 2) * (Wp + 2), C)
    extra = R - (Hp + 2) * (Wp + 2)
    if extra:
        y = jnp.pad(y, ((0, 0), (0, extra), (0, 0)))
    return y


def _conv_layer(x_flat, w_taps, scale, shift, *, nb, Wp2, Ho, Wo, R):
    B, _, Cin = x_flat.shape
    Cout = w_taps.shape[2]
    Hp, Wp = Ho // 2, Wo // 2
    NB = _pick_nb(B, nb)
    return pl.pallas_call(
        functools.partial(_conv_kernel, Wp2=Wp2, Ho=Ho, Wo=Wo),
        out_shape=jax.ShapeDtypeStruct((B, Hp * Wp, Cout), jnp.bfloat16),
        grid=(B // NB,),
        in_specs=[
            pl.BlockSpec((NB, R, Cin), lambda i: (i, 0, 0)),
            pl.BlockSpec((9, Cin, Cout), lambda i: (0, 0, 0)),
            pl.BlockSpec((1, Cout), lambda i: (0, 0)),
            pl.BlockSpec((1, Cout), lambda i: (0, 0)),
        ],
        out_specs=pl.BlockSpec((NB, Hp * Wp, Cout), lambda i: (i, 0, 0)),
        compiler_params=pltpu.CompilerParams(
            dimension_semantics=("parallel",),
            vmem_limit_bytes=VMEM_LIMIT,
        ),
    )(x_flat, w_taps, scale, shift)


def _l5_fc_kernel(x_ref, w_ref, s5_ref, b5_ref, wfc_ref, sfc_ref, bfc_ref,
                  out_ref):
    NB = x_ref.shape[0]
    acc = jnp.zeros((NB * 8, 1024), jnp.float32)
    t = 0
    for dh in range(3):
        for dw in range(3):
            off = dh * 4 + dw
            lhs = x_ref[:, off:off + 8, :].reshape(NB * 8, 256)
            acc = acc + jnp.dot(lhs, w_ref[t], preferred_element_type=jnp.float32)
            t += 1
    z = jnp.maximum(acc * s5_ref[...] + b5_ref[...], 0.0)
    z = jnp.max(z.reshape(NB, 2, 4, 1024), axis=1)
    z = jnp.max(z[:, :2, :], axis=1)
    f = jnp.dot(z.astype(jnp.bfloat16), wfc_ref[...],
                preferred_element_type=jnp.float32)
    f = jnp.maximum(f * sfc_ref[...] + bfc_ref[...], 0.0)
    out_ref[...] = f[:, :3600]


def _layer5_fc(x_flat, w_taps, s5, b5, wfc, sfc, bfc):
    B = x_flat.shape[0]
    NB = _pick_nb(B, 128)
    return pl.pallas_call(
        _l5_fc_kernel,
        out_shape=jax.ShapeDtypeStruct((B, 3600), jnp.float32),
        grid=(B // NB,),
        in_specs=[
            pl.BlockSpec((NB, 24, 256), lambda i: (i, 0, 0)),
            pl.BlockSpec((9, 256, 1024), lambda i: (0, 0, 0)),
            pl.BlockSpec((1, 1024), lambda i: (0, 0)),
            pl.BlockSpec((1, 1024), lambda i: (0, 0)),
            pl.BlockSpec((1024, 3840), lambda i: (0, 0)),
            pl.BlockSpec((1, 3840), lambda i: (0, 0)),
            pl.BlockSpec((1, 3840), lambda i: (0, 0)),
        ],
        out_specs=pl.BlockSpec((NB, 3600), lambda i: (i, 0)),
        compiler_params=pltpu.CompilerParams(
            dimension_semantics=("parallel",),
            vmem_limit_bytes=VMEM_LIMIT,
        ),
    )(x_flat, w_taps, s5, b5, wfc, sfc, bfc)


def kernel(x_nchw,
           layer1_w, layer1_scale, layer1_shift,
           layer2_w, layer2_scale, layer2_shift,
           layer3_w, layer3_scale, layer3_shift,
           layer4_w, layer4_scale, layer4_shift,
           layer5_w, layer5_scale, layer5_shift,
           fc1_w, fc1_scale, fc1_shift):
    y = _layer1(x_nchw, layer1_w, layer1_scale, layer1_shift)
    y = _conv3_layer(_kw3_input(y, 16, 16, 64),
                     _w_kw3(layer2_w, 64, 128), layer2_scale, layer2_shift,
                     nb=64, Ho=16, Wo=16)
    y = _conv3_layer(_kw3_input(y, 8, 8, 128),
                     _w_kw3(layer3_w, 128, 256), layer3_scale, layer3_shift,
                     nb=128, Ho=8, Wo=8)
    y = _conv_layer(_pad_flat(y, 4, 4, 256, 40),
                    _taps(layer4_w, 256, 256), layer4_scale, layer4_shift,
                    nb=256, Wp2=6, Ho=4, Wo=4, R=40)
    return _layer5_fc(_pad_flat(y, 2, 2, 256, 24),
                      _taps(layer5_w, 256, 1024), layer5_scale, layer5_shift,
                      fc1_w, fc1_scale, fc1_shift)


# speedup vs baseline: 4.2641x; 1.2963x over previous
import functools

import jax
import jax.numpy as jnp
from jax.experimental import pallas as pl
from jax.experimental.pallas import tpu as pltpu

VMEM_LIMIT = 64 * 1024 * 1024


def _pick_nb(B, nb):
    nb = min(nb, B)
    while B % nb:
        nb //= 2
    return max(nb, 1)


def _l1_kernel(xp_ref, wb_ref, scale_ref, shift_ref, out_ref):
    NB = xp_ref.shape[0]
    acc = jnp.zeros((NB * 32, 2048), jnp.float32)
    for p in range(6):
        lhs = xp_ref[:, 2 * p:2 * p + 32, :].reshape(NB * 32, 256)
        acc = acc + jnp.dot(lhs, wb_ref[p], preferred_element_type=jnp.float32)
    z = jnp.maximum(acc * scale_ref[...] + shift_ref[...], 0.0)
    z = z.reshape(NB, 32, 32, 64)
    z = jnp.max(z.reshape(NB, 16, 2, 32, 64), axis=2)
    z = jnp.max(z.reshape(NB, 16, 16, 2, 64), axis=3)
    out_ref[...] = z.reshape(NB, 256, 64).astype(out_ref.dtype)


def _build_l1_band(layer1_w):
    w4 = layer1_w[:363].astype(jnp.float32).reshape(3, 11, 11, 64)
    wi = jnp.arange(42)
    wo = jnp.arange(32)
    kw = jnp.arange(11)
    band = (wi[None, :, None] - wo[None, None, :] == kw[:, None, None])
    band = band.astype(jnp.float32)
    wb = jnp.einsum("kiw,cdkn->dicwn", band, w4)
    wb = wb.reshape(11, 126, 2048)
    wb = jnp.pad(wb, ((0, 1), (0, 2), (0, 0)))
    return wb.reshape(6, 256, 2048).astype(jnp.bfloat16)


def _layer1(x_nchw, layer1_w, scale, shift):
    B = x_nchw.shape[0]
    x = jnp.transpose(x_nchw, (0, 2, 3, 1)).astype(jnp.bfloat16)
    x = jnp.pad(x, ((0, 0), (1, 1), (1, 1), (0, 0)))
    x2 = jnp.pad(x.reshape(B, 42, 126), ((0, 0), (0, 0), (0, 2)))
    xshift = jnp.pad(x2[:, 1:, :], ((0, 0), (0, 1), (0, 0)))
    xpair = jnp.concatenate([x2, xshift], axis=2)
    wb = _build_l1_band(layer1_w)
    s = jnp.tile(scale, (1, 32))
    t = jnp.tile(shift, (1, 32))
    NB = _pick_nb(B, 64)
    return pl.pallas_call(
        _l1_kernel,
        out_shape=jax.ShapeDtypeStruct((B, 256, 64), jnp.bfloat16),
        grid=(B // NB,),
        in_specs=[
            pl.BlockSpec((NB, 42, 256), lambda i: (i, 0, 0)),
            pl.BlockSpec((6, 256, 2048), lambda i: (0, 0, 0)),
            pl.BlockSpec((1, 2048), lambda i: (0, 0)),
            pl.BlockSpec((1, 2048), lambda i: (0, 0)),
        ],
        out_specs=pl.BlockSpec((NB, 256, 64), lambda i: (i, 0, 0)),
        compiler_params=pltpu.CompilerParams(
            dimension_semantics=("parallel",),
            vmem_limit_bytes=VMEM_LIMIT,
        ),
    )(xpair, wb, s, t)


def _conv3_body(x_ref, w_ref, scale_ref, shift_ref, *, Hin, Win):
    NB, _, C = x_ref.shape
    x4 = x_ref[...].reshape(NB, Hin, Win, C)
    x4 = jnp.pad(x4, ((0, 0), (1, 1), (1, 1), (0, 0)))
    xs = jnp.concatenate([x4[:, :, dw:dw + Win, :] for dw in range(3)],
                         axis=3)
    xs = xs.reshape(NB, (Hin + 2) * Win, 3 * C)
    M2 = Hin * Win
    Cout = w_ref.shape[2]
    acc = jnp.zeros((NB * M2, Cout), jnp.float32)
    for dh in range(3):
        lhs = xs[:, dh * Win:dh * Win + M2, :].reshape(NB * M2, 3 * C)
        acc = acc + jnp.dot(lhs, w_ref[dh], preferred_element_type=jnp.float32)
    return jnp.maximum(acc * scale_ref[...] + shift_ref[...], 0.0)


def _conv3_kernel(x_ref, w_ref, scale_ref, shift_ref, out_ref, *, Hin, Win):
    NB = x_ref.shape[0]
    Cout = w_ref.shape[2]
    z = _conv3_body(x_ref, w_ref, scale_ref, shift_ref, Hin=Hin, Win=Win)
    Hp, Wp = Hin // 2, Win // 2
    z = jnp.max(z.reshape(NB, Hp, 2, Win, Cout), axis=2)
    z = jnp.max(z.reshape(NB, Hp, Wp, 2, Cout), axis=3)
    out_ref[...] = z.reshape(NB, Hp * Wp, Cout).astype(out_ref.dtype)


def _w_kw3(w_flat, cin, cout):
    w = w_flat[:cin * 9].reshape(cin, 3, 3, cout)
    return w.transpose(1, 2, 0, 3).reshape(3, 3 * cin, cout)


def _conv3_layer(y, w3, scale, shift, *, nb, Hin, Win):
    B = y.shape[0]
    C = y.shape[2]
    Cout = w3.shape[2]
    Hp, Wp = Hin // 2, Win // 2
    NB = _pick_nb(B, nb)
    return pl.pallas_call(
        functools.partial(_conv3_kernel, Hin=Hin, Win=Win),
        out_shape=jax.ShapeDtypeStruct((B, Hp * Wp, Cout), jnp.bfloat16),
        grid=(B // NB,),
        in_specs=[
            pl.BlockSpec((NB, Hin * Win, C), lambda i: (i, 0, 0)),
            pl.BlockSpec((3, 3 * C, Cout), lambda i: (0, 0, 0)),
            pl.BlockSpec((1, Cout), lambda i: (0, 0)),
            pl.BlockSpec((1, Cout), lambda i: (0, 0)),
        ],
        out_specs=pl.BlockSpec((NB, Hp * Wp, Cout), lambda i: (i, 0, 0)),
        compiler_params=pltpu.CompilerParams(
            dimension_semantics=("parallel",),
            vmem_limit_bytes=VMEM_LIMIT,
        ),
    )(y, w3, scale, shift)


def _l5_fc_kernel(x_ref, w_ref, s5_ref, b5_ref, wfc_ref, sfc_ref, bfc_ref,
                  out_ref):
    NB = x_ref.shape[0]
    z = _conv3_body(x_ref, w_ref, s5_ref, b5_ref, Hin=2, Win=2)
    z = jnp.max(z.reshape(NB, 2, 2, 1024), axis=(1, 2))
    f = jnp.dot(z.astype(jnp.bfloat16), wfc_ref[...],
                preferred_element_type=jnp.float32)
    f = jnp.maximum(f * sfc_ref[...] + bfc_ref[...], 0.0)
    out_ref[...] = f[:, :3600]


def _layer5_fc(y, w3, s5, b5, wfc, sfc, bfc):
    B = y.shape[0]
    NB = _pick_nb(B, 128)
    return pl.pallas_call(
        _l5_fc_kernel,
        out_shape=jax.ShapeDtypeStruct((B, 3600), jnp.float32),
        grid=(B // NB,),
        in_specs=[
            pl.BlockSpec((NB, 4, 256), lambda i: (i, 0, 0)),
            pl.BlockSpec((3, 768, 1024), lambda i: (0, 0, 0)),
            pl.BlockSpec((1, 1024), lambda i: (0, 0)),
            pl.BlockSpec((1, 1024), lambda i: (0, 0)),
            pl.BlockSpec((1024, 3840), lambda i: (0, 0)),
            pl.BlockSpec((1, 3840), lambda i: (0, 0)),
            pl.BlockSpec((1, 3840), lambda i: (0, 0)),
        ],
        out_specs=pl.BlockSpec((NB, 3600), lambda i: (i, 0)),
        compiler_params=pltpu.CompilerParams(
            dimension_semantics=("parallel",),
            vmem_limit_bytes=VMEM_LIMIT,
        ),
    )(y, w3, s5, b5, wfc, sfc, bfc)


def kernel(x_nchw,
           layer1_w, layer1_scale, layer1_shift,
           layer2_w, layer2_scale, layer2_shift,
           layer3_w, layer3_scale, layer3_shift,
           layer4_w, layer4_scale, layer4_shift,
           layer5_w, layer5_scale, layer5_shift,
           fc1_w, fc1_scale, fc1_shift):
    y = _layer1(x_nchw, layer1_w, layer1_scale, layer1_shift)
    y = _conv3_layer(y, _w_kw3(layer2_w, 64, 128),
                     layer2_scale, layer2_shift,
                     nb=64, Hin=16, Win=16)
    y = _conv3_layer(y, _w_kw3(layer3_w, 128, 256),
                     layer3_scale, layer3_shift,
                     nb=128, Hin=8, Win=8)
    y = _conv3_layer(y, _w_kw3(layer4_w, 256, 256),
                     layer4_scale, layer4_shift,
                     nb=256, Hin=4, Win=4)
    return _layer5_fc(y, _w_kw3(layer5_w, 256, 1024),
                      layer5_scale, layer5_shift,
                      fc1_w, fc1_scale, fc1_shift)


# speedup vs baseline: 4.3205x; 1.0132x over previous
import functools

import jax
import jax.numpy as jnp
from jax.experimental import pallas as pl
from jax.experimental.pallas import tpu as pltpu

VMEM_LIMIT = 64 * 1024 * 1024


def _pick_nb(B, nb):
    nb = min(nb, B)
    while B % nb:
        nb //= 2
    return max(nb, 1)


def _l1_kernel(xp_ref, wb_ref, scale_ref, shift_ref, out_ref):
    NB = xp_ref.shape[0]
    x2 = xp_ref[...]
    sh = jnp.concatenate(
        [x2[:, 1:, :], jnp.zeros((NB, 1, 128), x2.dtype)], axis=1)
    xp = jnp.concatenate([x2, sh], axis=2)
    acc = jnp.zeros((NB * 32, 2048), jnp.float32)
    for p in range(6):
        lhs = xp[:, 2 * p:2 * p + 32, :].reshape(NB * 32, 256)
        acc = acc + jnp.dot(lhs, wb_ref[p], preferred_element_type=jnp.float32)
    z = jnp.maximum(acc * scale_ref[...] + shift_ref[...], 0.0)
    z = z.reshape(NB, 32, 32, 64)
    z = jnp.max(z.reshape(NB, 16, 2, 32, 64), axis=2)
    z = jnp.max(z.reshape(NB, 16, 16, 2, 64), axis=3)
    out_ref[...] = z.reshape(NB, 256, 64).astype(out_ref.dtype)


def _build_l1_band(layer1_w):
    w4 = layer1_w[:363].astype(jnp.float32).reshape(3, 11, 11, 64)
    wi = jnp.arange(42)
    wo = jnp.arange(32)
    kw = jnp.arange(11)
    band = (wi[None, :, None] - wo[None, None, :] == kw[:, None, None])
    band = band.astype(jnp.float32)
    wb = jnp.einsum("kiw,cdkn->dicwn", band, w4)
    wb = wb.reshape(11, 126, 2048)
    wb = jnp.pad(wb, ((0, 1), (0, 2), (0, 0)))
    return wb.reshape(6, 256, 2048).astype(jnp.bfloat16)


def _layer1(x_nchw, layer1_w, scale, shift):
    B = x_nchw.shape[0]
    x = jnp.transpose(x_nchw, (0, 2, 3, 1)).astype(jnp.bfloat16)
    x = jnp.pad(x, ((0, 0), (1, 1), (1, 1), (0, 0)))
    x2 = jnp.pad(x.reshape(B, 42, 126), ((0, 0), (0, 0), (0, 2)))
    wb = _build_l1_band(layer1_w)
    s = jnp.tile(scale, (1, 32))
    t = jnp.tile(shift, (1, 32))
    NB = _pick_nb(B, 64)
    return pl.pallas_call(
        _l1_kernel,
        out_shape=jax.ShapeDtypeStruct((B, 256, 64), jnp.bfloat16),
        grid=(B // NB,),
        in_specs=[
            pl.BlockSpec((NB, 42, 128), lambda i: (i, 0, 0)),
            pl.BlockSpec((6, 256, 2048), lambda i: (0, 0, 0)),
            pl.BlockSpec((1, 2048), lambda i: (0, 0)),
            pl.BlockSpec((1, 2048), lambda i: (0, 0)),
        ],
        out_specs=pl.BlockSpec((NB, 256, 64), lambda i: (i, 0, 0)),
        compiler_params=pltpu.CompilerParams(
            dimension_semantics=("parallel",),
            vmem_limit_bytes=VMEM_LIMIT,
        ),
    )(x2, wb, s, t)


def _conv3_body(x_ref, w_ref, scale_ref, shift_ref, *, Hin, Win):
    NB, _, C = x_ref.shape
    x4 = x_ref[...].reshape(NB, Hin, Win, C)
    x4 = jnp.pad(x4, ((0, 0), (1, 1), (1, 1), (0, 0)))
    xs = jnp.concatenate([x4[:, :, dw:dw + Win, :] for dw in range(3)],
                         axis=3)
    xs = xs.reshape(NB, (Hin + 2) * Win, 3 * C)
    M2 = Hin * Win
    Cout = w_ref.shape[2]
    acc = jnp.zeros((NB * M2, Cout), jnp.float32)
    for dh in range(3):
        lhs = xs[:, dh * Win:dh * Win + M2, :].reshape(NB * M2, 3 * C)
        acc = acc + jnp.dot(lhs, w_ref[dh], preferred_element_type=jnp.float32)
    return jnp.maximum(acc * scale_ref[...] + shift_ref[...], 0.0)


def _conv3_kernel(x_ref, w_ref, scale_ref, shift_ref, out_ref, *, Hin, Win):
    NB = x_ref.shape[0]
    Cout = w_ref.shape[2]
    z = _conv3_body(x_ref, w_ref, scale_ref, shift_ref, Hin=Hin, Win=Win)
    Hp, Wp = Hin // 2, Win // 2
    z = jnp.max(z.reshape(NB, Hp, 2, Win, Cout), axis=2)
    z = jnp.max(z.reshape(NB, Hp, Wp, 2, Cout), axis=3)
    out_ref[...] = z.reshape(NB, Hp * Wp, Cout).astype(out_ref.dtype)


def _w_kw3(w_flat, cin, cout):
    w = w_flat[:cin * 9].reshape(cin, 3, 3, cout)
    return w.transpose(1, 2, 0, 3).reshape(3, 3 * cin, cout)


def _conv3_layer(y, w3, scale, shift, *, nb, Hin, Win):
    B = y.shape[0]
    C = y.shape[2]
    Cout = w3.shape[2]
    Hp, Wp = Hin // 2, Win // 2
    NB = _pick_nb(B, nb)
    return pl.pallas_call(
        functools.partial(_conv3_kernel, Hin=Hin, Win=Win),
        out_shape=jax.ShapeDtypeStruct((B, Hp * Wp, Cout), jnp.bfloat16),
        grid=(B // NB,),
        in_specs=[
            pl.BlockSpec((NB, Hin * Win, C), lambda i: (i, 0, 0)),
            pl.BlockSpec((3, 3 * C, Cout), lambda i: (0, 0, 0)),
            pl.BlockSpec((1, Cout), lambda i: (0, 0)),
            pl.BlockSpec((1, Cout), lambda i: (0, 0)),
        ],
        out_specs=pl.BlockSpec((NB, Hp * Wp, Cout), lambda i: (i, 0, 0)),
        compiler_params=pltpu.CompilerParams(
            dimension_semantics=("parallel",),
            vmem_limit_bytes=VMEM_LIMIT,
        ),
    )(y, w3, scale, shift)


def _l5_fc_kernel(x_ref, w_ref, s5_ref, b5_ref, wfc_ref, sfc_ref, bfc_ref,
                  out_ref):
    NB = x_ref.shape[0]
    z = _conv3_body(x_ref, w_ref, s5_ref, b5_ref, Hin=2, Win=2)
    z = jnp.max(z.reshape(NB, 2, 2, 1024), axis=(1, 2))
    f = jnp.dot(z.astype(jnp.bfloat16), wfc_ref[...],
                preferred_element_type=jnp.float32)
    f = jnp.maximum(f * sfc_ref[...] + bfc_ref[...], 0.0)
    out_ref[...] = f[:, :3600]


def _layer5_fc(y, w3, s5, b5, wfc, sfc, bfc):
    B = y.shape[0]
    NB = _pick_nb(B, 128)
    return pl.pallas_call(
        _l5_fc_kernel,
        out_shape=jax.ShapeDtypeStruct((B, 3600), jnp.float32),
        grid=(B // NB,),
        in_specs=[
            pl.BlockSpec((NB, 4, 256), lambda i: (i, 0, 0)),
            pl.BlockSpec((3, 768, 1024), lambda i: (0, 0, 0)),
            pl.BlockSpec((1, 1024), lambda i: (0, 0)),
            pl.BlockSpec((1, 1024), lambda i: (0, 0)),
            pl.BlockSpec((1024, 3840), lambda i: (0, 0)),
            pl.BlockSpec((1, 3840), lambda i: (0, 0)),
            pl.BlockSpec((1, 3840), lambda i: (0, 0)),
        ],
        out_specs=pl.BlockSpec((NB, 3600), lambda i: (i, 0)),
        compiler_params=pltpu.CompilerParams(
            dimension_semantics=("parallel",),
            vmem_limit_bytes=VMEM_LIMIT,
        ),
    )(y, w3, s5, b5, wfc, sfc, bfc)


def kernel(x_nchw,
           layer1_w, layer1_scale, layer1_shift,
           layer2_w, layer2_scale, layer2_shift,
           layer3_w, layer3_scale, layer3_shift,
           layer4_w, layer4_scale, layer4_shift,
           layer5_w, layer5_scale, layer5_shift,
           fc1_w, fc1_scale, fc1_shift):
    y = _layer1(x_nchw, layer1_w, layer1_scale, layer1_shift)
    y = _conv3_layer(y, _w_kw3(layer2_w, 64, 128),
                     layer2_scale, layer2_shift,
                     nb=64, Hin=16, Win=16)
    y = _conv3_layer(y, _w_kw3(layer3_w, 128, 256),
                     layer3_scale, layer3_shift,
                     nb=128, Hin=8, Win=8)
    y = _conv3_layer(y, _w_kw3(layer4_w, 256, 256),
                     layer4_scale, layer4_shift,
                     nb=256, Hin=4, Win=4)
    return _layer5_fc(y, _w_kw3(layer5_w, 256, 1024),
                      layer5_scale, layer5_shift,
                      fc1_w, fc1_scale, fc1_shift)
